# baseline (speedup 1.0000x reference)
import functools

import jax
import jax.numpy as jnp
from jax import lax
from jax.experimental import pallas as pl
from jax.experimental.pallas import tpu as pltpu
from jax.experimental.pallas import tpu_sc as plsc

BS, W, D, NSAMP = 16384, 10, 64, 2
V = 1000000
N = BS * W * NSAMP
NWORK = 32
PER = N // NWORK
CH = 256
NCH = PER // CH
SUB = 128
NSUB = CH // SUB
IDXROWS = PER // SUB
PAD = 17
KK = D // 16

_mesh = plsc.VectorSubcoreMesh(core_axis_name="c", subcore_axis_name="s")


@functools.partial(
    pl.kernel,
    mesh=_mesh,
    out_type=jax.ShapeDtypeStruct((N,), jnp.float32),
    compiler_params=pltpu.CompilerParams(
        needs_layout_passes=False, use_tc_tiling_on_sc=False
    ),
    scratch_types=[
        pltpu.VMEM((IDXROWS, SUB), jnp.int32),
        pltpu.VMEM((CH, D), jnp.float32),
        pltpu.VMEM((CH, D), jnp.float32),
        pltpu.VMEM((CH, D), jnp.float32),
        pltpu.VMEM((CH, D), jnp.float32),
        pltpu.VMEM((CH, PAD), jnp.float32),
        pltpu.VMEM((CH,), jnp.float32),
        pltpu.SemaphoreType.DMA,
        pltpu.SemaphoreType.DMA,
    ],
)
def _sc_dots(emb_hbm, cent_hbm, sidx_hbm, out_hbm, idx_v, rows0_v, rows1_v,
             cent0_v, cent1_v, part_v, dots_v, sem0, sem1):
    cid = lax.axis_index("c")
    sid = lax.axis_index("s")
    wid = sid * 2 + cid
    base = wid * PER
    lane = lax.broadcasted_iota(jnp.int32, (16,), 0)
    rows = (rows0_v, rows1_v)
    cent = (cent0_v, cent1_v)
    sems = (sem0, sem1)

    pltpu.sync_copy(sidx_hbm.at[pl.ds(wid * IDXROWS, IDXROWS)], idx_v)

    def issue(ph, ci):
        s0 = base + ci * CH
        b0 = jnp.bitwise_and(s0, BS - 1)
        pltpu.async_copy(cent_hbm.at[pl.ds(b0, CH)], cent[ph], sems[ph])
        for j in range(NSUB):
            pltpu.async_copy(emb_hbm.at[idx_v.at[ci * NSUB + j]],
                             rows[ph].at[pl.ds(j * SUB, SUB)], sems[ph])

    def drain(ph):
        pltpu.make_async_copy(cent_hbm.at[pl.ds(0, CH)], cent[ph],
                              sems[ph]).wait()
        for j in range(NSUB):
            pltpu.make_async_copy(emb_hbm.at[idx_v.at[j]],
                                  rows[ph].at[pl.ds(j * SUB, SUB)],
                                  sems[ph]).wait()

    def compute(ph, ci):
        s0 = base + ci * CH

        def phase_a(g, c2):
            for j in range(8):
                s = g * 8 + j
                acc = rows[ph][s, pl.ds(0, 16)] * cent[ph][s, pl.ds(0, 16)]
                for kk in range(1, KK):
                    acc = acc + (rows[ph][s, pl.ds(kk * 16, 16)]
                                 * cent[ph][s, pl.ds(kk * 16, 16)])
                part_v[s, pl.ds(0, 16)] = acc
            return c2

        lax.fori_loop(0, CH // 8, phase_a, 0)

        def phase_b(g, c2):
            rows16 = g * 16 + lane
            acc = plsc.load_gather(part_v, [rows16, jnp.zeros((16,), jnp.int32)])
            for k in range(1, 16):
                acc = acc + plsc.load_gather(
                    part_v, [rows16, jnp.full((16,), k, jnp.int32)])
            plsc.store_scatter(dots_v, [rows16], acc)
            return c2

        lax.fori_loop(0, CH // 16, phase_b, 0)
        pltpu.sync_copy(dots_v, out_hbm.at[pl.ds(s0, CH)])

    issue(0, 0)

    def pair_body(i, carry):
        ci0 = 2 * i
        drain(0)
        issue(1, ci0 + 1)
        compute(0, ci0)
        drain(1)

        @pl.when(ci0 + 2 < NCH)
        def _():
            issue(0, ci0 + 2)

        compute(1, ci0 + 1)
        return carry

    lax.fori_loop(0, NCH // 2, pair_body, 0)


TBLK = 16384
TGRID = (V + TBLK - 1) // TBLK
VP = TGRID * TBLK
THALF = TBLK // 2


def _tc_transpose_body(in_ref, out_ref):
    t = jnp.transpose(in_ref[...])
    out_ref[...] = jnp.concatenate(
        [t[0:THALF], t[THALF:TBLK]], axis=1)


def _tc_transpose(emb_t):
    return pl.pallas_call(
        _tc_transpose_body,
        grid=(TGRID,),
        in_specs=[pl.BlockSpec((D, TBLK), lambda g: (0, g))],
        out_specs=pl.BlockSpec((THALF, 2 * D), lambda g: (g, 0)),
        out_shape=jax.ShapeDtypeStruct((VP // 2, 2 * D), jnp.float32),
    )(emb_t)


def _softplus(x):
    return jnp.maximum(x, 0.0) + jnp.log1p(jnp.exp(-jnp.abs(x)))


BLK = 2048
GRID = BS // BLK
DROWS = N // 128 // GRID


def _tc_loss_body(ctx_ref, cent_ref, dots_ref, out_ref):
    i = pl.program_id(0)
    ce = cent_ref[...]
    part = jnp.sum(_softplus(dots_ref[...])) / N
    for w in range(W):
        ts = jnp.sum(ctx_ref[w, :, :] * ce, axis=0, keepdims=True)
        part = part + jnp.sum(_softplus(-ts)) / (BS * W)

    @pl.when(i == 0)
    def _():
        out_ref[0, 0] = 0.0

    out_ref[0, 0] += part


def _tc_loss(ctx_t, cent_t, dots2d):
    return pl.pallas_call(
        _tc_loss_body,
        grid=(GRID,),
        in_specs=[
            pl.BlockSpec((W, D, BLK), lambda i: (0, 0, i)),
            pl.BlockSpec((D, BLK), lambda i: (0, i)),
            pl.BlockSpec((DROWS, 128), lambda i: (i, 0)),
        ],
        out_specs=pl.BlockSpec((1, 1), lambda i: (0, 0),
                               memory_space=pltpu.SMEM),
        out_shape=jax.ShapeDtypeStruct((1, 1), jnp.float32),
    )(ctx_t, cent_t, dots2d)


def kernel(center, context, emb_table, sample_idx):
    r = jnp.transpose(sample_idx, (1, 2, 0)).astype(jnp.int32)
    lb = TBLK.bit_length() - 1
    vidx = ((r >> lb) << lb) + 2 * jnp.bitwise_and(r, THALF - 1) \
        + jnp.bitwise_and(r >> (lb - 1), 1)
    sidx = vidx.reshape(N // SUB, SUB)
    emb2 = _tc_transpose(jnp.transpose(emb_table, (1, 0)))
    dots = _sc_dots(emb2.reshape(VP, D), center, sidx)
    ctx_t = jnp.transpose(context, (1, 2, 0))
    cent_t = jnp.transpose(center, (1, 0))
    out = _tc_loss(ctx_t, cent_t, dots.reshape(N // 128, 128))
    return out[0, 0]


# speedup vs baseline: 1.0414x; 1.0414x over previous
import functools

import jax
import jax.numpy as jnp
from jax import lax
from jax.experimental import pallas as pl
from jax.experimental.pallas import tpu as pltpu
from jax.experimental.pallas import tpu_sc as plsc

BS, W, D, NSAMP = 16384, 10, 64, 2
V = 1000000
N = BS * W * NSAMP
NWORK = 32
PER = N // NWORK
CH = 256
NCH = PER // CH
SUB = 128
NSUB = CH // SUB
IDXROWS = PER // SUB
PAD = 17
KK = D // 16

_mesh = plsc.VectorSubcoreMesh(core_axis_name="c", subcore_axis_name="s")


@functools.partial(
    pl.kernel,
    mesh=_mesh,
    out_type=jax.ShapeDtypeStruct((N,), jnp.float32),
    compiler_params=pltpu.CompilerParams(
        needs_layout_passes=False, use_tc_tiling_on_sc=False
    ),
    scratch_types=[
        pltpu.VMEM((IDXROWS, SUB), jnp.int32),
        pltpu.VMEM((CH, D), jnp.float32),
        pltpu.VMEM((CH, D), jnp.float32),
        pltpu.VMEM((CH, D), jnp.float32),
        pltpu.VMEM((CH, D), jnp.float32),
        pltpu.VMEM((CH, PAD), jnp.float32),
        pltpu.VMEM((CH,), jnp.float32),
        pltpu.SemaphoreType.DMA,
        pltpu.SemaphoreType.DMA,
    ],
)
def _sc_dots(emb_hbm, cent_hbm, sidx_hbm, out_hbm, idx_v, rows0_v, rows1_v,
             cent0_v, cent1_v, part_v, dots_v, sem0, sem1):
    cid = lax.axis_index("c")
    sid = lax.axis_index("s")
    wid = sid * 2 + cid
    base = wid * PER
    lane = lax.broadcasted_iota(jnp.int32, (16,), 0)
    rows = (rows0_v, rows1_v)
    cent = (cent0_v, cent1_v)
    sems = (sem0, sem1)

    pltpu.sync_copy(sidx_hbm.at[pl.ds(wid * IDXROWS, IDXROWS)], idx_v)

    def issue(ph, ci):
        s0 = base + ci * CH
        b0 = jnp.bitwise_and(s0, BS - 1)
        pltpu.async_copy(cent_hbm.at[pl.ds(b0, CH)], cent[ph], sems[ph])
        for j in range(NSUB):
            pltpu.async_copy(emb_hbm.at[idx_v.at[ci * NSUB + j]],
                             rows[ph].at[pl.ds(j * SUB, SUB)], sems[ph])

    def drain(ph):
        pltpu.make_async_copy(cent_hbm.at[pl.ds(0, CH)], cent[ph],
                              sems[ph]).wait()
        for j in range(NSUB):
            pltpu.make_async_copy(emb_hbm.at[idx_v.at[j]],
                                  rows[ph].at[pl.ds(j * SUB, SUB)],
                                  sems[ph]).wait()

    def compute(ph, ci):
        s0 = base + ci * CH

        def phase_a(g, c2):
            for j in range(8):
                s = g * 8 + j
                acc = rows[ph][s, pl.ds(0, 16)] * cent[ph][s, pl.ds(0, 16)]
                for kk in range(1, KK):
                    acc = acc + (rows[ph][s, pl.ds(kk * 16, 16)]
                                 * cent[ph][s, pl.ds(kk * 16, 16)])
                part_v[s, pl.ds(0, 16)] = acc
            return c2

        lax.fori_loop(0, CH // 8, phase_a, 0)

        def phase_b(g, c2):
            rows16 = g * 16 + lane
            acc = plsc.load_gather(part_v, [rows16, jnp.zeros((16,), jnp.int32)])
            for k in range(1, 16):
                acc = acc + plsc.load_gather(
                    part_v, [rows16, jnp.full((16,), k, jnp.int32)])
            plsc.store_scatter(dots_v, [rows16], acc)
            return c2

        lax.fori_loop(0, CH // 16, phase_b, 0)
        pltpu.sync_copy(dots_v, out_hbm.at[pl.ds(s0, CH)])

    issue(0, 0)

    def pair_body(i, carry):
        ci0 = 2 * i
        drain(0)
        issue(1, ci0 + 1)
        compute(0, ci0)
        drain(1)

        @pl.when(ci0 + 2 < NCH)
        def _():
            issue(0, ci0 + 2)

        compute(1, ci0 + 1)
        return carry

    lax.fori_loop(0, NCH // 2, pair_body, 0)


TBLK = 32768
TGRID = (V + TBLK - 1) // TBLK
VP = TGRID * TBLK
THALF = TBLK // 2


def _tc_transpose_body(in_ref, out_ref):
    t = jnp.transpose(in_ref[...])
    out_ref[...] = jnp.concatenate(
        [t[0:THALF], t[THALF:TBLK]], axis=1)


def _tc_transpose(emb_t):
    return pl.pallas_call(
        _tc_transpose_body,
        grid=(TGRID,),
        in_specs=[pl.BlockSpec((D, TBLK), lambda g: (0, g))],
        out_specs=pl.BlockSpec((THALF, 2 * D), lambda g: (g, 0)),
        out_shape=jax.ShapeDtypeStruct((VP // 2, 2 * D), jnp.float32),
    )(emb_t)


def _softplus(x):
    return jnp.maximum(x, 0.0) + jnp.log1p(jnp.exp(-jnp.abs(x)))


BLK = 2048
GRID = BS // BLK
DROWS = N // 128 // GRID


def _tc_loss_body(ctx_ref, cent_ref, dots_ref, out_ref):
    i = pl.program_id(0)
    ce = cent_ref[...]
    part = jnp.sum(_softplus(dots_ref[...])) / N
    for w in range(W):
        ts = jnp.sum(ctx_ref[w, :, :] * ce, axis=0, keepdims=True)
        part = part + jnp.sum(_softplus(-ts)) / (BS * W)

    @pl.when(i == 0)
    def _():
        out_ref[0, 0] = 0.0

    out_ref[0, 0] += part


def _tc_loss(ctx_t, cent_t, dots2d):
    return pl.pallas_call(
        _tc_loss_body,
        grid=(GRID,),
        in_specs=[
            pl.BlockSpec((W, D, BLK), lambda i: (0, 0, i)),
            pl.BlockSpec((D, BLK), lambda i: (0, i)),
            pl.BlockSpec((DROWS, 128), lambda i: (i, 0)),
        ],
        out_specs=pl.BlockSpec((1, 1), lambda i: (0, 0),
                               memory_space=pltpu.SMEM),
        out_shape=jax.ShapeDtypeStruct((1, 1), jnp.float32),
    )(ctx_t, cent_t, dots2d)


def kernel(center, context, emb_table, sample_idx):
    r = jnp.transpose(sample_idx, (1, 2, 0)).astype(jnp.int32)
    lb = TBLK.bit_length() - 1
    vidx = ((r >> lb) << lb) + 2 * jnp.bitwise_and(r, THALF - 1) \
        + jnp.bitwise_and(r >> (lb - 1), 1)
    sidx = vidx.reshape(N // SUB, SUB)
    emb2 = _tc_transpose(jnp.transpose(emb_table, (1, 0)))
    dots = _sc_dots(emb2.reshape(VP, D), center, sidx)
    ctx_t = jnp.transpose(context, (1, 2, 0))
    cent_t = jnp.transpose(center, (1, 0))
    out = _tc_loss(ctx_t, cent_t, dots.reshape(N // 128, 128))
    return out[0, 0]
